# baseline (speedup 1.0000x reference)
import functools

import jax
import jax.numpy as jnp
from jax import lax
from jax.experimental import pallas as pl
from jax.experimental.pallas import tpu as pltpu
from jax.experimental.pallas import tpu_sc as plsc

NUM_EMB = 1000000
DIM = 64
BATCH = 16384

_NW = 32
_CHUNK = 512
_NCH = 1953
_TAIL = _NCH * _CHUNK
_PIECE = 2048
_WCAP = 2048
_FLUSH = 64
_OUT_ROWS = BATCH + 64


def _build():
    mesh = plsc.VectorSubcoreMesh(core_axis_name="c", subcore_axis_name="s")

    @functools.partial(
        pl.kernel,
        mesh=mesh,
        out_type=jax.ShapeDtypeStruct((_OUT_ROWS, 128), jnp.float32),
        scratch_types=[
            pltpu.VMEM((_PIECE,), jnp.int32),
            pltpu.VMEM((BATCH,), jnp.int32),
            pltpu.VMEM((BATCH,), jnp.int32),
            pltpu.VMEM((_WCAP,), jnp.int32),
            pltpu.VMEM((_WCAP,), jnp.int32),
            pltpu.VMEM((DIM, _CHUNK), jnp.float32),
            pltpu.VMEM((DIM, _CHUNK), jnp.float32),
            pltpu.VMEM((80, 128), jnp.float32),
            pltpu.VMEM((_FLUSH,), jnp.int32),
            pltpu.SemaphoreType.DMA,
            pltpu.SemaphoreType.DMA,
        ],
        compiler_params=pltpu.CompilerParams(needs_layout_passes=False),
    )
    def gather_kernel(
        table_t_hbm, ids_hbm, out_hbm,
        piece_v, me_v, mp_v, we_v, wp_v, chunk0_v, chunk1_v, brows_v, bidx_v,
        sem0, sem1,
    ):
        wid = lax.axis_index("s") * 2 + lax.axis_index("c")
        lanes = lax.iota(jnp.int32, 16)

        def start_chunk(c, buf, sem):
            g = c * _NW + wid
            coff = pl.multiple_of(g * _CHUNK, 128)
            pltpu.async_copy(table_t_hbm.at[:, pl.ds(coff, _CHUNK)], buf, sem)

        ncw = jnp.where(wid < _NCH % _NW, _NCH // _NW + 1, _NCH // _NW)

        start_chunk(0, chunk0_v, sem0)
        start_chunk(1, chunk1_v, sem1)

        def flush(rc):
            sl = rc + lanes
            sm = sl < _FLUSH
            plsc.store_scatter(
                bidx_v, [jnp.minimum(sl, _FLUSH - 1)], BATCH + lanes, mask=sm
            )
            pltpu.sync_copy(brows_v.at[pl.ds(0, _FLUSH)], out_hbm.at[bidx_v])
            return jnp.int32(0)

        def scan_piece(s, mcnt):
            pltpu.sync_copy(ids_hbm.at[pl.ds(s * _PIECE, _PIECE)], piece_v)

            def scan_vec(i, mcnt):
                e = plsc.load_gather(piece_v, [i * 16 + lanes])
                pos = s * _PIECE + i * 16 + lanes
                m = (e >> 9) % _NW == wid
                mi = m.astype(jnp.int32)
                rank = mcnt + plsc.cumsum(mi) - 1
                plsc.store_scatter(me_v, [rank], e, mask=m)
                plsc.store_scatter(mp_v, [rank], pos, mask=m)
                return mcnt + jnp.sum(mi)

            return lax.fori_loop(0, _PIECE // 16, scan_vec, mcnt)

        mcnt = lax.fori_loop(0, BATCH // _PIECE, scan_piece, jnp.int32(0))
        mvecs = (mcnt + 15) >> 4

        def process_chunk(c, buf, rc):
            g = c * _NW + wid
            coff = g * _CHUNK

            def rescan(i, carry):
                cm, skip = carry
                k = i * 16 + lanes
                kc = jnp.minimum(k, BATCH - 1)
                e = plsc.load_gather(me_v, [kc])
                p = plsc.load_gather(mp_v, [kc])
                m = (k < mcnt) & ((e >> 9) == g)
                mi = m.astype(jnp.int32)
                widx = cm + plsc.cumsum(mi) - 1 - skip
                wm = m & (widx >= 0) & (widx < _WCAP)
                plsc.store_scatter(we_v, [widx], e, mask=wm)
                plsc.store_scatter(wp_v, [widx], p, mask=wm)
                return cm + jnp.sum(mi), skip

            def do_groups(wcnt, rc):
                def group(j, rc):
                    rc = lax.cond(rc >= _FLUSH - 16, flush, lambda r: r, rc)
                    k = j * 16 + lanes
                    kv = k < wcnt
                    kc = jnp.minimum(k, _WCAP - 1)
                    e = plsc.load_gather(we_v, [kc])
                    p = plsc.load_gather(wp_v, [kc])
                    e_loc = jnp.where(kv, e - coff, 0)
                    kvi = kv.astype(jnp.int32)
                    slot = jnp.where(kv, rc + plsc.cumsum(kvi) - 1, 64 + lanes)
                    plsc.store_scatter(
                        bidx_v, [jnp.minimum(slot, _FLUSH - 1)], p, mask=kv
                    )
                    for f in range(DIM):
                        fv = jnp.full((16,), f, jnp.int32)
                        vals = plsc.load_gather(buf, [fv, e_loc])
                        plsc.store_scatter(brows_v, [slot, fv], vals)
                    return rc + jnp.sum(kvi)

                return lax.fori_loop(0, (wcnt + 15) >> 4, group, rc)

            cm, _ = lax.fori_loop(0, mvecs, rescan, (jnp.int32(0), jnp.int32(0)))
            rc = do_groups(jnp.minimum(cm, _WCAP), rc)

            def extra_pass(carry):
                skip, rc = carry
                lax.fori_loop(0, mvecs, rescan, (jnp.int32(0), skip))
                rc = do_groups(jnp.minimum(cm - skip, _WCAP), rc)
                return skip + _WCAP, rc

            def more(carry):
                skip, _ = carry
                return skip < cm

            _, rc = lax.while_loop(more, extra_pass, (jnp.int32(_WCAP), rc))
            return rc

        def pair(p, rc):
            for par, buf, sem in ((0, chunk0_v, sem0), (1, chunk1_v, sem1)):
                c = 2 * p + par

                def run(rc, buf=buf, sem=sem, c=c):
                    pltpu.make_async_copy(
                        table_t_hbm.at[:, pl.ds(0, _CHUNK)], buf, sem
                    ).wait()
                    new_rc = process_chunk(c, buf, rc)

                    @pl.when(c + 2 < ncw)
                    def _():
                        start_chunk(c + 2, buf, sem)

                    return new_rc

                rc = lax.cond(c < ncw, run, lambda r: r, rc)
            return rc

        rc = lax.fori_loop(0, (_NCH // _NW + 2) // 2, pair, jnp.int32(0))

        def pad_strip(q, carry):
            sl = rc + q * 16 + lanes
            sm = sl < _FLUSH
            plsc.store_scatter(
                bidx_v, [jnp.minimum(sl, _FLUSH - 1)], BATCH + q * 16 + lanes,
                mask=sm,
            )
            return carry

        @pl.when(rc > 0)
        def _():
            lax.fori_loop(0, 4, pad_strip, jnp.int32(0))
            pltpu.sync_copy(brows_v.at[pl.ds(0, _FLUSH)], out_hbm.at[bidx_v])

    return gather_kernel


_GATHER = _build()


def kernel(ids, table):
    ids32 = ids.astype(jnp.int32)
    out_w = _GATHER(table.T, ids32)
    out = out_w[:BATCH, :DIM]
    tail = lax.slice(table, (_TAIL, 0), (NUM_EMB, DIM))
    tvals = jnp.take(tail, jnp.clip(ids32 - _TAIL, 0, NUM_EMB - _TAIL - 1), axis=0)
    return jnp.where((ids32 >= _TAIL)[:, None], tvals, out)


# speedup vs baseline: 1.0255x; 1.0255x over previous
import functools

import jax
import jax.numpy as jnp
from jax import lax
from jax.experimental import pallas as pl
from jax.experimental.pallas import tpu as pltpu
from jax.experimental.pallas import tpu_sc as plsc

NUM_EMB = 1000000
DIM = 64
BATCH = 16384

_NW = 32
_CHUNK = 512
_NCH = 1953
_TAIL = _NCH * _CHUNK
_PIECE = 2048
_WCAP = 2048
_FLUSH = 64
_OUT_ROWS = BATCH + 64


def _build():
    mesh = plsc.VectorSubcoreMesh(core_axis_name="c", subcore_axis_name="s")

    @functools.partial(
        pl.kernel,
        mesh=mesh,
        out_type=jax.ShapeDtypeStruct((_OUT_ROWS, 128), jnp.float32),
        scratch_types=[
            pltpu.VMEM((_PIECE,), jnp.int32),
            pltpu.VMEM((BATCH + 16,), jnp.int32),
            pltpu.VMEM((BATCH + 16,), jnp.int32),
            pltpu.VMEM((_WCAP,), jnp.int32),
            pltpu.VMEM((_WCAP,), jnp.int32),
            pltpu.VMEM((DIM, _CHUNK), jnp.float32),
            pltpu.VMEM((DIM, _CHUNK), jnp.float32),
            pltpu.VMEM((80, 128), jnp.float32),
            pltpu.VMEM((_FLUSH,), jnp.int32),
            pltpu.SemaphoreType.DMA,
            pltpu.SemaphoreType.DMA,
        ],
        compiler_params=pltpu.CompilerParams(needs_layout_passes=False),
    )
    def gather_kernel(
        table_t_hbm, ids_hbm, out_hbm,
        piece_v, me_v, mp_v, we_v, wp_v, chunk0_v, chunk1_v, brows_v, bidx_v,
        sem0, sem1,
    ):
        wid = lax.axis_index("s") * 2 + lax.axis_index("c")
        lanes = lax.iota(jnp.int32, 16)

        def start_chunk(c, buf, sem):
            g = c * _NW + wid
            coff = pl.multiple_of(g * _CHUNK, 128)
            pltpu.async_copy(table_t_hbm.at[:, pl.ds(coff, _CHUNK)], buf, sem)

        ncw = jnp.where(wid < _NCH % _NW, _NCH // _NW + 1, _NCH // _NW)

        start_chunk(0, chunk0_v, sem0)
        start_chunk(1, chunk1_v, sem1)

        def flush(rc):
            sl = rc + lanes
            sm = sl < _FLUSH
            plsc.store_scatter(
                bidx_v, [jnp.minimum(sl, _FLUSH - 1)], BATCH + lanes, mask=sm
            )
            pltpu.sync_copy(brows_v.at[pl.ds(0, _FLUSH)], out_hbm.at[bidx_v])
            return jnp.int32(0)

        def scan_piece(s, mcnt):
            pltpu.sync_copy(ids_hbm.at[pl.ds(s * _PIECE, _PIECE)], piece_v)

            def scan_vec(i, mcnt):
                e = plsc.load_gather(piece_v, [i * 16 + lanes])
                pos = s * _PIECE + i * 16 + lanes
                m = (e >> 9) % _NW == wid
                plsc.store_compressed(me_v.at[pl.ds(mcnt, 16)], e, mask=m)
                plsc.store_compressed(mp_v.at[pl.ds(mcnt, 16)], pos, mask=m)
                cnt = plsc.all_reduce_population_count(m)
                return mcnt + cnt[0]

            return lax.fori_loop(0, _PIECE // 16, scan_vec, mcnt)

        mcnt = lax.fori_loop(0, BATCH // _PIECE, scan_piece, jnp.int32(0))
        mvecs = (mcnt + 15) >> 4

        def process_chunk(c, buf, rc):
            g = c * _NW + wid
            coff = g * _CHUNK

            def rescan(i, carry):
                cm, skip = carry
                k = i * 16 + lanes
                kc = jnp.minimum(k, BATCH - 1)
                e = plsc.load_gather(me_v, [kc])
                p = plsc.load_gather(mp_v, [kc])
                m = (k < mcnt) & ((e >> 9) == g)
                mi = m.astype(jnp.int32)
                cs = plsc.cumsum(mi)
                widx = cm + cs - 1 - skip
                wm = m & (widx >= 0) & (widx < _WCAP)
                plsc.store_scatter(we_v, [widx], e, mask=wm)
                plsc.store_scatter(wp_v, [widx], p, mask=wm)
                return cm + cs[15], skip

            def do_groups(wcnt, rc):
                def group(j, rc):
                    rc = lax.cond(rc >= _FLUSH - 16, flush, lambda r: r, rc)
                    k = j * 16 + lanes
                    kv = k < wcnt
                    kc = jnp.minimum(k, _WCAP - 1)
                    e = plsc.load_gather(we_v, [kc])
                    p = plsc.load_gather(wp_v, [kc])
                    e_loc = jnp.where(kv, e - coff, 0)
                    kvi = kv.astype(jnp.int32)
                    kcs = plsc.cumsum(kvi)
                    slot = jnp.where(kv, rc + kcs - 1, 64 + lanes)
                    plsc.store_scatter(
                        bidx_v, [jnp.minimum(slot, _FLUSH - 1)], p, mask=kv
                    )
                    for f in range(DIM):
                        fv = jnp.full((16,), f, jnp.int32)
                        vals = plsc.load_gather(buf, [fv, e_loc])
                        plsc.store_scatter(brows_v, [slot, fv], vals)
                    return rc + kcs[15]

                return lax.fori_loop(0, (wcnt + 15) >> 4, group, rc)

            cm, _ = lax.fori_loop(0, mvecs, rescan, (jnp.int32(0), jnp.int32(0)))
            rc = do_groups(jnp.minimum(cm, _WCAP), rc)

            def extra_pass(carry):
                skip, rc = carry
                lax.fori_loop(0, mvecs, rescan, (jnp.int32(0), skip))
                rc = do_groups(jnp.minimum(cm - skip, _WCAP), rc)
                return skip + _WCAP, rc

            def more(carry):
                skip, _ = carry
                return skip < cm

            _, rc = lax.while_loop(more, extra_pass, (jnp.int32(_WCAP), rc))
            return rc

        def pair(p, rc):
            for par, buf, sem in ((0, chunk0_v, sem0), (1, chunk1_v, sem1)):
                c = 2 * p + par

                def run(rc, buf=buf, sem=sem, c=c):
                    pltpu.make_async_copy(
                        table_t_hbm.at[:, pl.ds(0, _CHUNK)], buf, sem
                    ).wait()
                    new_rc = process_chunk(c, buf, rc)

                    @pl.when(c + 2 < ncw)
                    def _():
                        start_chunk(c + 2, buf, sem)

                    return new_rc

                rc = lax.cond(c < ncw, run, lambda r: r, rc)
            return rc

        rc = lax.fori_loop(0, (_NCH // _NW + 2) // 2, pair, jnp.int32(0))

        def pad_strip(q, carry):
            sl = rc + q * 16 + lanes
            sm = sl < _FLUSH
            plsc.store_scatter(
                bidx_v, [jnp.minimum(sl, _FLUSH - 1)], BATCH + q * 16 + lanes,
                mask=sm,
            )
            return carry

        @pl.when(rc > 0)
        def _():
            lax.fori_loop(0, 4, pad_strip, jnp.int32(0))
            pltpu.sync_copy(brows_v.at[pl.ds(0, _FLUSH)], out_hbm.at[bidx_v])

    return gather_kernel


_GATHER = _build()


def kernel(ids, table):
    ids32 = ids.astype(jnp.int32)
    out_w = _GATHER(table.T, ids32)
    out = out_w[:BATCH, :DIM]
    tail = lax.slice(table, (_TAIL, 0), (NUM_EMB, DIM))
    tvals = jnp.take(tail, jnp.clip(ids32 - _TAIL, 0, NUM_EMB - _TAIL - 1), axis=0)
    return jnp.where((ids32 >= _TAIL)[:, None], tvals, out)


# speedup vs baseline: 1.0347x; 1.0089x over previous
import functools

import jax
import jax.numpy as jnp
from jax import lax
from jax.experimental import pallas as pl
from jax.experimental.pallas import tpu as pltpu
from jax.experimental.pallas import tpu_sc as plsc

NUM_EMB = 1000000
DIM = 64
BATCH = 16384

_NW = 32
_CHUNK = 512
_NCH = 1953
_TAIL = _NCH * _CHUNK
_PIECE = 2048
_WCAP = 2048
_FLUSH = 64
_OUT_ROWS = BATCH + 128


def _build():
    mesh = plsc.VectorSubcoreMesh(core_axis_name="c", subcore_axis_name="s")

    @functools.partial(
        pl.kernel,
        mesh=mesh,
        out_type=jax.ShapeDtypeStruct((_OUT_ROWS, 128), jnp.float32),
        scratch_types=[
            pltpu.VMEM((_PIECE,), jnp.int32),
            pltpu.VMEM((BATCH + 16,), jnp.int32),
            pltpu.VMEM((_WCAP,), jnp.int32),
            pltpu.VMEM((DIM, _CHUNK), jnp.float32),
            pltpu.VMEM((DIM, _CHUNK), jnp.float32),
            pltpu.VMEM((DIM, _CHUNK), jnp.float32),
            pltpu.VMEM((80, 128), jnp.float32),
            pltpu.VMEM((_FLUSH,), jnp.int32),
            pltpu.SemaphoreType.DMA,
            pltpu.SemaphoreType.DMA,
            pltpu.SemaphoreType.DMA,
        ],
        compiler_params=pltpu.CompilerParams(needs_layout_passes=False),
    )
    def gather_kernel(
        table_t_hbm, ids_hbm, out_hbm,
        piece_v, mw_v, wm_v, chunk0_v, chunk1_v, chunk2_v, brows_v, bidx_v,
        sem0, sem1, sem2,
    ):
        wid = lax.axis_index("s") * 2 + lax.axis_index("c")
        lanes = lax.iota(jnp.int32, 16)

        def start_chunk(c, buf, sem):
            g = c * _NW + wid
            coff = pl.multiple_of(g * _CHUNK, 128)
            pltpu.async_copy(table_t_hbm.at[:, pl.ds(coff, _CHUNK)], buf, sem)

        ncw = jnp.where(wid < _NCH % _NW, _NCH // _NW + 1, _NCH // _NW)

        start_chunk(0, chunk0_v, sem0)
        start_chunk(1, chunk1_v, sem1)
        start_chunk(2, chunk2_v, sem2)

        def flush(rc):
            sl = rc + lanes
            sm = sl < _FLUSH
            plsc.store_scatter(
                bidx_v, [jnp.minimum(sl, _FLUSH - 1)], BATCH + lanes, mask=sm
            )
            pltpu.sync_copy(brows_v.at[pl.ds(0, _FLUSH)], out_hbm.at[bidx_v])
            return jnp.int32(0)

        def scan_piece(s, mcnt):
            pltpu.sync_copy(ids_hbm.at[pl.ds(s * _PIECE, _PIECE)], piece_v)

            def scan_vec(i, mcnt):
                e = plsc.load_gather(piece_v, [i * 16 + lanes])
                pos = s * _PIECE + i * 16 + lanes
                g = e >> 9
                m = g % _NW == wid
                packed = ((g >> 5) << 23) | ((e & (_CHUNK - 1)) << 14) | pos
                plsc.store_compressed(mw_v.at[pl.ds(mcnt, 16)], packed, mask=m)
                cnt = plsc.all_reduce_population_count(m)
                return mcnt + cnt[0]

            return lax.fori_loop(0, _PIECE // 16, scan_vec, mcnt)

        mcnt = lax.fori_loop(0, BATCH // _PIECE, scan_piece, jnp.int32(0))
        mvecs = (mcnt + 15) >> 4

        def process_chunk(c, buf, sem, rc):
            g = c * _NW + wid
            coff = g * _CHUNK

            def rescan(i, carry):
                cm, skip = carry
                k = i * 16 + lanes
                kc = jnp.minimum(k, BATCH - 1)
                w = plsc.load_gather(mw_v, [kc])
                m = (k < mcnt) & ((w >> 23) == c)
                mi = m.astype(jnp.int32)
                cs = plsc.cumsum(mi)
                widx = cm + cs - 1 - skip
                ok = m & (widx >= 0) & (widx < _WCAP)
                plsc.store_scatter(wm_v, [widx], w, mask=ok)
                return cm + cs[15], skip

            def do_groups(wcnt, rc):
                def group(j, rc):
                    rc = lax.cond(rc >= _FLUSH - 16, flush, lambda r: r, rc)
                    k = j * 16 + lanes
                    kv = k < wcnt
                    kc = jnp.minimum(k, _WCAP - 1)
                    w = plsc.load_gather(wm_v, [kc])
                    p = w & 16383
                    e_loc = jnp.where(kv, (w >> 14) & (_CHUNK - 1), 0)
                    kvi = kv.astype(jnp.int32)
                    kcs = plsc.cumsum(kvi)
                    slot = jnp.where(kv, rc + kcs - 1, _FLUSH + lanes)
                    plsc.store_scatter(
                        bidx_v, [jnp.minimum(slot, _FLUSH - 1)], p, mask=kv
                    )
                    for f in range(DIM):
                        fv = jnp.full((16,), f, jnp.int32)
                        vals = plsc.load_gather(buf, [fv, e_loc])
                        plsc.store_scatter(brows_v, [slot, fv], vals)
                    return rc + kcs[15]

                return lax.fori_loop(0, (wcnt + 15) >> 4, group, rc)

            cm, _ = lax.fori_loop(0, mvecs, rescan, (jnp.int32(0), jnp.int32(0)))
            pltpu.make_async_copy(
                table_t_hbm.at[:, pl.ds(0, _CHUNK)], buf, sem
            ).wait()
            rc = do_groups(jnp.minimum(cm, _WCAP), rc)

            def extra_pass(carry):
                skip, rc = carry
                lax.fori_loop(0, mvecs, rescan, (jnp.int32(0), skip))
                rc = do_groups(jnp.minimum(cm - skip, _WCAP), rc)
                return skip + _WCAP, rc

            def more(carry):
                skip, _ = carry
                return skip < cm

            _, rc = lax.while_loop(more, extra_pass, (jnp.int32(_WCAP), rc))
            return rc

        def triple(p, rc):
            for par, buf, sem in (
                (0, chunk0_v, sem0), (1, chunk1_v, sem1), (2, chunk2_v, sem2)
            ):
                c = 3 * p + par

                def run(rc, buf=buf, sem=sem, c=c):
                    new_rc = process_chunk(c, buf, sem, rc)

                    @pl.when(c + 3 < ncw)
                    def _():
                        start_chunk(c + 3, buf, sem)

                    return new_rc

                rc = lax.cond(c < ncw, run, lambda r: r, rc)
            return rc

        rc = lax.fori_loop(0, (_NCH // _NW + 3) // 3, triple, jnp.int32(0))

        def pad_strip(q, carry):
            sl = rc + q * 16 + lanes
            sm = sl < _FLUSH
            plsc.store_scatter(
                bidx_v, [jnp.minimum(sl, _FLUSH - 1)], BATCH + q * 16 + lanes,
                mask=sm,
            )
            return carry

        @pl.when(rc > 0)
        def _():
            lax.fori_loop(0, _FLUSH // 16, pad_strip, jnp.int32(0))
            pltpu.sync_copy(brows_v.at[pl.ds(0, _FLUSH)], out_hbm.at[bidx_v])

    return gather_kernel


_GATHER = _build()


def kernel(ids, table):
    ids32 = ids.astype(jnp.int32)
    out_w = _GATHER(table.T, ids32)
    out = out_w[:BATCH, :DIM]
    tail = lax.slice(table, (_TAIL, 0), (NUM_EMB, DIM))
    tvals = jnp.take(tail, jnp.clip(ids32 - _TAIL, 0, NUM_EMB - _TAIL - 1), axis=0)
    return jnp.where((ids32 >= _TAIL)[:, None], tvals, out)


# speedup vs baseline: 1.1099x; 1.0727x over previous
import functools

import jax
import jax.numpy as jnp
from jax import lax
from jax.experimental import pallas as pl
from jax.experimental.pallas import tpu as pltpu
from jax.experimental.pallas import tpu_sc as plsc

NUM_EMB = 1000000
DIM = 64
BATCH = 16384

_NW = 32
_CHUNK = 512
_NCH = 1953
_TAIL = _NCH * _CHUNK
_PIECE = 2048
_WCAP = 2048
_FLUSH = 128
_OUT_ROWS = BATCH + 128


def _build():
    mesh = plsc.VectorSubcoreMesh(core_axis_name="c", subcore_axis_name="s")

    @functools.partial(
        pl.kernel,
        mesh=mesh,
        out_type=jax.ShapeDtypeStruct((_OUT_ROWS, 128), jnp.float32),
        scratch_types=[
            pltpu.VMEM((_PIECE,), jnp.int32),
            pltpu.VMEM((BATCH + 16,), jnp.int32),
            pltpu.VMEM((BATCH + 16,), jnp.int32),
            pltpu.VMEM((_WCAP,), jnp.int32),
            pltpu.VMEM((_WCAP,), jnp.int32),
            pltpu.VMEM((DIM, _CHUNK), jnp.float32),
            pltpu.VMEM((DIM, _CHUNK), jnp.float32),
            pltpu.VMEM((144, 128), jnp.float32),
            pltpu.VMEM((_FLUSH,), jnp.int32),
            pltpu.SemaphoreType.DMA,
            pltpu.SemaphoreType.DMA,
        ],
        compiler_params=pltpu.CompilerParams(needs_layout_passes=False),
    )
    def gather_kernel(
        table_t_hbm, ids_hbm, out_hbm,
        piece_v, me_v, mp_v, we_v, wp_v, chunk0_v, chunk1_v, brows_v, bidx_v,
        sem0, sem1,
    ):
        wid = lax.axis_index("s") * 2 + lax.axis_index("c")
        lanes = lax.iota(jnp.int32, 16)

        def start_chunk(c, buf, sem):
            g = c * _NW + wid
            coff = pl.multiple_of(g * _CHUNK, 128)
            pltpu.async_copy(table_t_hbm.at[:, pl.ds(coff, _CHUNK)], buf, sem)

        ncw = jnp.where(wid < _NCH % _NW, _NCH // _NW + 1, _NCH // _NW)

        start_chunk(0, chunk0_v, sem0)
        start_chunk(1, chunk1_v, sem1)

        def flush(rc):
            sl = rc + lanes
            sm = sl < _FLUSH
            plsc.store_scatter(
                bidx_v, [jnp.minimum(sl, _FLUSH - 1)], BATCH + lanes, mask=sm
            )
            pltpu.sync_copy(brows_v.at[pl.ds(0, _FLUSH)], out_hbm.at[bidx_v])
            return jnp.int32(0)

        def scan_piece(s, mcnt):
            pltpu.sync_copy(ids_hbm.at[pl.ds(s * _PIECE, _PIECE)], piece_v)

            def scan_vec(i, mcnt):
                e = plsc.load_gather(piece_v, [i * 16 + lanes])
                pos = s * _PIECE + i * 16 + lanes
                m = (e >> 9) % _NW == wid
                plsc.store_compressed(me_v.at[pl.ds(mcnt, 16)], e, mask=m)
                plsc.store_compressed(mp_v.at[pl.ds(mcnt, 16)], pos, mask=m)
                cnt = plsc.all_reduce_population_count(m)
                return mcnt + cnt[0]

            return lax.fori_loop(0, _PIECE // 16, scan_vec, mcnt)

        mcnt = lax.fori_loop(0, BATCH // _PIECE, scan_piece, jnp.int32(0))
        mvecs = (mcnt + 15) >> 4

        def process_chunk(c, buf, sem, rc):
            g = c * _NW + wid
            coff = g * _CHUNK

            def rescan(i, carry):
                cm, skip = carry
                k = i * 16 + lanes
                kc = jnp.minimum(k, BATCH - 1)
                e = plsc.load_gather(me_v, [kc])
                p = plsc.load_gather(mp_v, [kc])
                m = (k < mcnt) & ((e >> 9) == g)
                mi = m.astype(jnp.int32)
                cs = plsc.cumsum(mi)
                widx = cm + cs - 1 - skip
                wm = m & (widx >= 0) & (widx < _WCAP)
                plsc.store_scatter(we_v, [widx], e, mask=wm)
                plsc.store_scatter(wp_v, [widx], p, mask=wm)
                return cm + cs[15], skip

            def do_groups(wcnt, rc):
                def group(j, rc):
                    rc = lax.cond(rc >= _FLUSH - 16, flush, lambda r: r, rc)
                    k = j * 16 + lanes
                    kv = k < wcnt
                    kc = jnp.minimum(k, _WCAP - 1)
                    e = plsc.load_gather(we_v, [kc])
                    p = plsc.load_gather(wp_v, [kc])
                    e_loc = jnp.where(kv, e - coff, 0)
                    kvi = kv.astype(jnp.int32)
                    kcs = plsc.cumsum(kvi)
                    slot = jnp.where(kv, rc + kcs - 1, _FLUSH + lanes)
                    plsc.store_scatter(
                        bidx_v, [jnp.minimum(slot, _FLUSH - 1)], p, mask=kv
                    )
                    for f in range(DIM):
                        fv = jnp.full((16,), f, jnp.int32)
                        vals = plsc.load_gather(buf, [fv, e_loc])
                        plsc.store_scatter(brows_v, [slot, fv], vals)
                    return rc + kcs[15]

                return lax.fori_loop(0, (wcnt + 15) >> 4, group, rc)

            cm, _ = lax.fori_loop(0, mvecs, rescan, (jnp.int32(0), jnp.int32(0)))
            pltpu.make_async_copy(
                table_t_hbm.at[:, pl.ds(0, _CHUNK)], buf, sem
            ).wait()
            rc = do_groups(jnp.minimum(cm, _WCAP), rc)

            def extra_pass(carry):
                skip, rc = carry
                lax.fori_loop(0, mvecs, rescan, (jnp.int32(0), skip))
                rc = do_groups(jnp.minimum(cm - skip, _WCAP), rc)
                return skip + _WCAP, rc

            def more(carry):
                skip, _ = carry
                return skip < cm

            _, rc = lax.while_loop(more, extra_pass, (jnp.int32(_WCAP), rc))
            return rc

        def pair(p, rc):
            for par, buf, sem in ((0, chunk0_v, sem0), (1, chunk1_v, sem1)):
                c = 2 * p + par

                def run(rc, buf=buf, sem=sem, c=c):
                    new_rc = process_chunk(c, buf, sem, rc)

                    @pl.when(c + 2 < ncw)
                    def _():
                        start_chunk(c + 2, buf, sem)

                    return new_rc

                rc = lax.cond(c < ncw, run, lambda r: r, rc)
            return rc

        rc = lax.fori_loop(0, (_NCH // _NW + 2) // 2, pair, jnp.int32(0))

        def pad_strip(q, carry):
            sl = rc + q * 16 + lanes
            sm = sl < _FLUSH
            plsc.store_scatter(
                bidx_v, [jnp.minimum(sl, _FLUSH - 1)], BATCH + q * 16 + lanes,
                mask=sm,
            )
            return carry

        @pl.when(rc > 0)
        def _():
            lax.fori_loop(0, _FLUSH // 16, pad_strip, jnp.int32(0))
            pltpu.sync_copy(brows_v.at[pl.ds(0, _FLUSH)], out_hbm.at[bidx_v])

    return gather_kernel


_GATHER = _build()


def kernel(ids, table):
    ids32 = ids.astype(jnp.int32)
    out_w = _GATHER(table.T, ids32)
    out = out_w[:BATCH, :DIM]
    tail = lax.slice(table, (_TAIL, 0), (NUM_EMB, DIM))
    tvals = jnp.take(tail, jnp.clip(ids32 - _TAIL, 0, NUM_EMB - _TAIL - 1), axis=0)
    return jnp.where((ids32 >= _TAIL)[:, None], tvals, out)
